# baseline (speedup 1.0000x reference)
import functools

import jax
import jax.numpy as jnp
from jax import lax
from jax.experimental import pallas as pl
from jax.experimental.pallas import tpu as pltpu
from jax.experimental.pallas import tpu_sc as plsc

_L = 16
_B = 1024
_NC = 2
_NS = 16
_SL = 8


def _digits(key):
    d1 = jnp.minimum(lax.shift_right_logical(key, 20), _B - 1)
    d2 = jnp.bitwise_and(lax.shift_right_logical(key, 10), _B - 1)
    d3 = jnp.bitwise_and(key, _B - 1)
    return d1, d2, d3


def _level_digit(w16, bsel, lvl):
    key = plsc.bitcast(w16, jnp.int32)
    d1, d2, d3 = _digits(key)
    if lvl == 0:
        return d1, w16
    if lvl == 1:
        act = d1 == bsel[0]
        return d2, jnp.where(act, w16, 0.0)
    act = (d1 == bsel[0]) & (d2 == bsel[1])
    return d3, jnp.where(act, w16, 0.0)


def _policy_sc(attn, thresh16):
    batch, n = attn.shape
    spw = n // _SL
    nchunk = spw // _L
    rows_per_core = batch // _NC
    mesh = plsc.VectorSubcoreMesh(
        core_axis_name="c", subcore_axis_name="s", num_cores=_NC,
        num_subcores=_NS)

    @functools.partial(
        pl.kernel,
        out_type=jax.ShapeDtypeStruct((batch, n), jnp.float32),
        mesh=mesh,
        compiler_params=pltpu.CompilerParams(needs_layout_passes=False),
        scratch_types=[
            pltpu.VMEM((spw,), jnp.float32),
            pltpu.VMEM((_L * _B,), jnp.float32),
            pltpu.VMEM((_B,), jnp.float32),
            pltpu.VMEM((spw,), jnp.float32),
            pltpu.VMEM((_L,), jnp.float32),
            pltpu.VMEM((_L,), jnp.float32),
            pltpu.VMEM((_SL, _L), jnp.float32),
            pltpu.VMEM((_SL, _B), jnp.float32),
            pltpu.VMEM_SHARED((rows_per_core, 3, _SL, _B), jnp.float32),
            pltpu.VMEM_SHARED((rows_per_core, _SL, _L), jnp.float32),
        ],
    )
    def k(attn_hbm, th_hbm, out_hbm, w_v, h_v, c_v, o_v, t_v, tmp_v, tb_v,
          r_v, sh_h, sh_t):
        cid = lax.axis_index("c")
        sid = lax.axis_index("s")
        rin = sid // _SL
        row = cid * rows_per_core + rin
        sl = sid % _SL
        off = sl * spw

        pltpu.sync_copy(attn_hbm.at[row, pl.ds(off, spw)], w_v)
        pltpu.sync_copy(th_hbm, t_v)
        thv = t_v[...]
        lane = lax.iota(jnp.int32, _L)
        idx15 = jnp.full((_L,), _L - 1, jnp.int32)
        zero16 = jnp.zeros((_L,), jnp.float32)
        slv = jnp.full((_L,), sl, jnp.int32)

        def zh(g, _):
            h_v[pl.ds(g * _L, _L)] = jnp.zeros((_L,), jnp.float32)
            return 0
        lax.fori_loop(0, _L * _B // _L, zh, 0)

        base = zero16
        bsel = []
        for lvl in range(3):
            def hbody(i, _, lvl=lvl):
                w16 = w_v[pl.ds(i * _L, _L)]
                d, val = _level_digit(w16, bsel, lvl)
                plsc.addupdate_scatter(h_v, [lane * _B + d], val)
                return 0
            lax.fori_loop(0, nchunk, hbody, 0)

            def rbody(g, _):
                acc = h_v[pl.ds(g * _L, _L)]
                for l in range(1, _L):
                    acc = acc + h_v[pl.ds(l * _B + g * _L, _L)]
                c_v[pl.ds(g * _L, _L)] = acc
                return 0
            lax.fori_loop(0, _B // _L, rbody, 0)
            pltpu.sync_copy(c_v, sh_h.at[rin, lvl, sl])

            def clbody(i, _, lvl=lvl):
                w16 = w_v[pl.ds(i * _L, _L)]
                d, _val = _level_digit(w16, bsel, lvl)
                plsc.store_scatter(h_v, [lane * _B + d],
                                   jnp.zeros((_L,), jnp.float32))
                return 0
            lax.fori_loop(0, nchunk, clbody, 0)
            plsc.subcore_barrier()

            pltpu.sync_copy(sh_h.at[rin, lvl], r_v)

            def cbody(g, carry):
                acc = r_v[0, pl.ds(g * _L, _L)]
                for kk in range(1, _SL):
                    acc = acc + r_v[kk, pl.ds(g * _L, _L)]
                cs = plsc.cumsum(acc) + carry
                c_v[pl.ds(g * _L, _L)] = cs
                return plsc.load_gather(
                    c_v, [jnp.full((_L,), g * _L + _L - 1, jnp.int32)])
            total = lax.fori_loop(0, _B // _L, cbody, zero16)

            remv = total - (thv - base)

            def kbody(g, cntl):
                c16 = c_v[pl.ds(g * _L, _L)]
                return cntl + (c16 <= remv).astype(jnp.float32)
            cntl = lax.fori_loop(0, _B // _L, kbody, zero16)
            tmp_v[...] = plsc.cumsum(cntl)
            cnt = plsc.load_gather(tmp_v, [idx15])
            bstar = jnp.minimum(cnt.astype(jnp.int32), _B - 1)

            cb = plsc.load_gather(c_v, [bstar])
            base = base + (total - cb)
            bsel.append(bstar)

        keystar = jnp.bitwise_or(
            jnp.bitwise_or(lax.shift_left(bsel[0], 20),
                           lax.shift_left(bsel[1], 10)), bsel[2])
        wstar16 = plsc.bitcast(keystar, jnp.float32)

        def tbody(i, acc):
            w16 = w_v[pl.ds(i * _L, _L)]
            key = plsc.bitcast(w16, jnp.int32)
            d1, d2, d3 = _digits(key)
            e3 = (d1 == bsel[0]) & (d2 == bsel[1]) & (d3 == bsel[2])
            return acc + e3.astype(jnp.float32)
        tcl = lax.fori_loop(0, nchunk, tbody, zero16)
        tmp_v[...] = plsc.cumsum(tcl)
        myties = plsc.load_gather(tmp_v, [idx15])
        tmp_v[...] = myties
        pltpu.sync_copy(tmp_v, sh_t.at[rin, sl])
        plsc.subcore_barrier()
        pltpu.sync_copy(sh_t.at[rin], tb_v)
        toff = zero16
        for kk in range(_SL):
            ck = tb_v[kk, :]
            toff = toff + jnp.where(jnp.full((_L,), kk, jnp.int32) < slv,
                                    ck, 0.0)

        def fbody(i, tcar):
            w16 = w_v[pl.ds(i * _L, _L)]
            key = plsc.bitcast(w16, jnp.int32)
            d1, d2, d3 = _digits(key)
            e1 = d1 == bsel[0]
            e2 = e1 & (d2 == bsel[1])
            e3 = e2 & (d3 == bsel[2])
            a3f = e3.astype(jnp.float32)
            t16 = plsc.cumsum(a3f) + tcar
            sel = ((d1 > bsel[0])
                   | (e1 & (d2 > bsel[1]))
                   | (e2 & (d3 > bsel[2]))
                   | (e3 & (base + t16 * wstar16 < thv)))
            o_v[pl.ds(i * _L, _L)] = 1.0 - sel.astype(jnp.float32)
            tmp_v[...] = t16
            return plsc.load_gather(tmp_v, [idx15])
        lax.fori_loop(0, nchunk, fbody, toff)
        pltpu.sync_copy(o_v, out_hbm.at[row, pl.ds(off, spw)])

    return k(attn, thresh16)


def _copy_tc(x2):
    m, d = x2.shape
    tb = 2048

    def body(x_ref, y_ref):
        y_ref[...] = x_ref[...]

    return pl.pallas_call(
        body,
        grid=(m // tb,),
        in_specs=[pl.BlockSpec((tb, d), lambda g: (g, 0))],
        out_specs=pl.BlockSpec((tb, d), lambda g: (g, 0)),
        out_shape=jax.ShapeDtypeStruct((m, d), x2.dtype),
    )(x2)


def _fixup_tc(y2, pol2):
    m, d = y2.shape
    tb = 1024
    nb = m // tb

    def body(y_in, p_ref, y_out, vbuf, sem):
        for i in range(nb):
            pol = p_ref[pl.ds(i * tb, tb), :]

            @pl.when(jnp.min(pol) < 0.5)
            def _(i=i, pol=pol):
                cp_in = pltpu.make_async_copy(
                    y_in.at[pl.ds(i * tb, tb), :], vbuf, sem)
                cp_in.start()
                cp_in.wait()
                vbuf[...] = vbuf[...] * pol
                cp_out = pltpu.make_async_copy(
                    vbuf, y_out.at[pl.ds(i * tb, tb), :], sem)
                cp_out.start()
                cp_out.wait()

    return pl.pallas_call(
        body,
        in_specs=[
            pl.BlockSpec(memory_space=pl.ANY),
            pl.BlockSpec(memory_space=pltpu.VMEM),
        ],
        out_specs=pl.BlockSpec(memory_space=pl.ANY),
        out_shape=jax.ShapeDtypeStruct((m, d), y2.dtype),
        scratch_shapes=[
            pltpu.VMEM((tb, d), jnp.float32),
            pltpu.SemaphoreType.DMA,
        ],
        input_output_aliases={0: 0},
    )(y2, pol2)


def kernel(x, attn_weight, thresh):
    batch, n, d = x.shape
    th16 = jnp.broadcast_to(thresh.astype(jnp.float32), (_L,))
    policy = _policy_sc(attn_weight, th16)
    y0 = _copy_tc(x.reshape(batch * n, d))
    y2 = _fixup_tc(y0, policy.reshape(batch * n, 1))
    return (y2.reshape(batch, n, d), policy)


# speedup vs baseline: 1.0279x; 1.0279x over previous
import functools

import jax
import jax.numpy as jnp
from jax import lax
from jax.experimental import pallas as pl
from jax.experimental.pallas import tpu as pltpu
from jax.experimental.pallas import tpu_sc as plsc

_L = 16
_B = 1024
_NC = 2
_NS = 16
_SL = 8


def _digits(key):
    d1 = jnp.minimum(lax.shift_right_logical(key, 20), _B - 1)
    d2 = jnp.bitwise_and(lax.shift_right_logical(key, 10), _B - 1)
    d3 = jnp.bitwise_and(key, _B - 1)
    return d1, d2, d3


def _level_digit(w16, bsel, lvl):
    key = plsc.bitcast(w16, jnp.int32)
    d1, d2, d3 = _digits(key)
    if lvl == 0:
        return d1, w16
    if lvl == 1:
        act = d1 == bsel[0]
        return d2, jnp.where(act, w16, 0.0)
    act = (d1 == bsel[0]) & (d2 == bsel[1])
    return d3, jnp.where(act, w16, 0.0)


def _policy_sc(attn, thresh16):
    batch, n = attn.shape
    spw = n // _SL
    nchunk = spw // _L
    rows_per_core = batch // _NC
    mesh = plsc.VectorSubcoreMesh(
        core_axis_name="c", subcore_axis_name="s", num_cores=_NC,
        num_subcores=_NS)

    @functools.partial(
        pl.kernel,
        out_type=jax.ShapeDtypeStruct((batch, n), jnp.float32),
        mesh=mesh,
        compiler_params=pltpu.CompilerParams(needs_layout_passes=False),
        scratch_types=[
            pltpu.VMEM((spw,), jnp.float32),
            pltpu.VMEM((_L * _B,), jnp.float32),
            pltpu.VMEM((_B,), jnp.float32),
            pltpu.VMEM((spw,), jnp.float32),
            pltpu.VMEM((_L,), jnp.float32),
            pltpu.VMEM((_L,), jnp.float32),
            pltpu.VMEM((_SL, _L), jnp.float32),
            pltpu.VMEM((_SL, _B), jnp.float32),
            pltpu.VMEM_SHARED((rows_per_core, 3, _SL, _B), jnp.float32),
            pltpu.VMEM_SHARED((rows_per_core, _SL, _L), jnp.float32),
        ],
    )
    def k(attn_hbm, th_hbm, out_hbm, w_v, h_v, c_v, o_v, t_v, tmp_v, tb_v,
          r_v, sh_h, sh_t):
        cid = lax.axis_index("c")
        sid = lax.axis_index("s")
        rin = sid // _SL
        row = cid * rows_per_core + rin
        sl = sid % _SL
        off = sl * spw

        pltpu.sync_copy(attn_hbm.at[row, pl.ds(off, spw)], w_v)
        pltpu.sync_copy(th_hbm, t_v)
        thv = t_v[...]
        lane = lax.iota(jnp.int32, _L)
        idx15 = jnp.full((_L,), _L - 1, jnp.int32)
        zero16 = jnp.zeros((_L,), jnp.float32)
        slv = jnp.full((_L,), sl, jnp.int32)

        def zh(g, _):
            h_v[pl.ds(g * _L, _L)] = jnp.zeros((_L,), jnp.float32)
            return 0
        lax.fori_loop(0, _L * _B // _L, zh, 0)

        base = zero16
        bsel = []
        for lvl in range(3):
            def hbody(i, _, lvl=lvl):
                w16 = w_v[pl.ds(i * _L, _L)]
                d, val = _level_digit(w16, bsel, lvl)
                plsc.addupdate_scatter(h_v, [lane * _B + d], val)
                return 0
            lax.fori_loop(0, nchunk, hbody, 0)

            def rbody(g, _):
                acc = h_v[pl.ds(g * _L, _L)]
                for l in range(1, _L):
                    acc = acc + h_v[pl.ds(l * _B + g * _L, _L)]
                c_v[pl.ds(g * _L, _L)] = acc
                return 0
            lax.fori_loop(0, _B // _L, rbody, 0)
            pltpu.sync_copy(c_v, sh_h.at[rin, lvl, sl])

            def clbody(i, _, lvl=lvl):
                w16 = w_v[pl.ds(i * _L, _L)]
                d, _val = _level_digit(w16, bsel, lvl)
                plsc.store_scatter(h_v, [lane * _B + d],
                                   jnp.zeros((_L,), jnp.float32))
                return 0
            lax.fori_loop(0, nchunk, clbody, 0)
            plsc.subcore_barrier()

            pltpu.sync_copy(sh_h.at[rin, lvl], r_v)

            def cbody(g, carry):
                acc = r_v[0, pl.ds(g * _L, _L)]
                for kk in range(1, _SL):
                    acc = acc + r_v[kk, pl.ds(g * _L, _L)]
                cs = plsc.cumsum(acc) + carry
                c_v[pl.ds(g * _L, _L)] = cs
                return plsc.load_gather(
                    c_v, [jnp.full((_L,), g * _L + _L - 1, jnp.int32)])
            total = lax.fori_loop(0, _B // _L, cbody, zero16)

            remv = total - (thv - base)

            def kbody(g, cntl):
                c16 = c_v[pl.ds(g * _L, _L)]
                return cntl + (c16 <= remv).astype(jnp.float32)
            cntl = lax.fori_loop(0, _B // _L, kbody, zero16)
            tmp_v[...] = plsc.cumsum(cntl)
            cnt = plsc.load_gather(tmp_v, [idx15])
            bstar = jnp.minimum(cnt.astype(jnp.int32), _B - 1)

            cb = plsc.load_gather(c_v, [bstar])
            base = base + (total - cb)
            bsel.append(bstar)

        keystar = jnp.bitwise_or(
            jnp.bitwise_or(lax.shift_left(bsel[0], 20),
                           lax.shift_left(bsel[1], 10)), bsel[2])
        wstar16 = plsc.bitcast(keystar, jnp.float32)

        def tbody(i, acc):
            w16 = w_v[pl.ds(i * _L, _L)]
            key = plsc.bitcast(w16, jnp.int32)
            d1, d2, d3 = _digits(key)
            e3 = (d1 == bsel[0]) & (d2 == bsel[1]) & (d3 == bsel[2])
            return acc + e3.astype(jnp.float32)
        tcl = lax.fori_loop(0, nchunk, tbody, zero16)
        tmp_v[...] = plsc.cumsum(tcl)
        myties = plsc.load_gather(tmp_v, [idx15])
        tmp_v[...] = myties
        pltpu.sync_copy(tmp_v, sh_t.at[rin, sl])
        plsc.subcore_barrier()
        pltpu.sync_copy(sh_t.at[rin], tb_v)
        toff = zero16
        for kk in range(_SL):
            ck = tb_v[kk, :]
            toff = toff + jnp.where(jnp.full((_L,), kk, jnp.int32) < slv,
                                    ck, 0.0)

        def fbody(i, tcar):
            w16 = w_v[pl.ds(i * _L, _L)]
            key = plsc.bitcast(w16, jnp.int32)
            d1, d2, d3 = _digits(key)
            e1 = d1 == bsel[0]
            e2 = e1 & (d2 == bsel[1])
            e3 = e2 & (d3 == bsel[2])
            a3f = e3.astype(jnp.float32)
            t16 = plsc.cumsum(a3f) + tcar
            sel = ((d1 > bsel[0])
                   | (e1 & (d2 > bsel[1]))
                   | (e2 & (d3 > bsel[2]))
                   | (e3 & (base + t16 * wstar16 < thv)))
            o_v[pl.ds(i * _L, _L)] = 1.0 - sel.astype(jnp.float32)
            tmp_v[...] = t16
            return plsc.load_gather(tmp_v, [idx15])
        lax.fori_loop(0, nchunk, fbody, toff)
        pltpu.sync_copy(o_v, out_hbm.at[row, pl.ds(off, spw)])

    return k(attn, thresh16)


def _copy_tc(x2):
    m, d = x2.shape
    nch = 16
    rows = m // nch
    nbuf = 4

    def body(x_ref, y_ref, bufs, insems, outsems):
        def start_in(i):
            pltpu.make_async_copy(
                x_ref.at[pl.ds(i * rows, rows), :],
                bufs.at[i % nbuf], insems.at[i % nbuf]).start()

        def wait_in(i):
            pltpu.make_async_copy(
                x_ref.at[pl.ds(i * rows, rows), :],
                bufs.at[i % nbuf], insems.at[i % nbuf]).wait()

        def start_out(i):
            pltpu.make_async_copy(
                bufs.at[i % nbuf],
                y_ref.at[pl.ds(i * rows, rows), :], outsems.at[i % nbuf]).start()

        def wait_out(i):
            pltpu.make_async_copy(
                bufs.at[i % nbuf],
                y_ref.at[pl.ds(i * rows, rows), :], outsems.at[i % nbuf]).wait()

        for i in range(nbuf):
            start_in(i)
        for i in range(nch):
            wait_in(i)
            start_out(i)
            if i + nbuf < nch:
                wait_out(i)
                start_in(i + nbuf)
        for i in range(max(nch - nbuf, 0), nch):
            wait_out(i)

    return pl.pallas_call(
        body,
        in_specs=[pl.BlockSpec(memory_space=pl.ANY)],
        out_specs=pl.BlockSpec(memory_space=pl.ANY),
        out_shape=jax.ShapeDtypeStruct((m, d), x2.dtype),
        scratch_shapes=[
            pltpu.VMEM((nbuf, rows, d), jnp.float32),
            pltpu.SemaphoreType.DMA((nbuf,)),
            pltpu.SemaphoreType.DMA((nbuf,)),
        ],
    )(x2)


def _fixup_tc(y2, pol2):
    m, d = y2.shape
    tb = 1024
    nb = m // tb

    def body(y_in, p_ref, y_out, vbuf, sem):
        for i in range(nb):
            pol = p_ref[pl.ds(i * tb, tb), :]

            @pl.when(jnp.min(pol) < 0.5)
            def _(i=i, pol=pol):
                cp_in = pltpu.make_async_copy(
                    y_in.at[pl.ds(i * tb, tb), :], vbuf, sem)
                cp_in.start()
                cp_in.wait()
                vbuf[...] = vbuf[...] * pol
                cp_out = pltpu.make_async_copy(
                    vbuf, y_out.at[pl.ds(i * tb, tb), :], sem)
                cp_out.start()
                cp_out.wait()

    return pl.pallas_call(
        body,
        in_specs=[
            pl.BlockSpec(memory_space=pl.ANY),
            pl.BlockSpec(memory_space=pltpu.VMEM),
        ],
        out_specs=pl.BlockSpec(memory_space=pl.ANY),
        out_shape=jax.ShapeDtypeStruct((m, d), y2.dtype),
        scratch_shapes=[
            pltpu.VMEM((tb, d), jnp.float32),
            pltpu.SemaphoreType.DMA,
        ],
        input_output_aliases={0: 0},
    )(y2, pol2)


def kernel(x, attn_weight, thresh):
    batch, n, d = x.shape
    th16 = jnp.broadcast_to(thresh.astype(jnp.float32), (_L,))
    policy = _policy_sc(attn_weight, th16)
    y0 = _copy_tc(x.reshape(batch * n, d))
    y2 = _fixup_tc(y0, policy.reshape(batch * n, 1))
    return (y2.reshape(batch, n, d), policy)


# speedup vs baseline: 1.0365x; 1.0084x over previous
import functools

import jax
import jax.numpy as jnp
from jax import lax
from jax.experimental import pallas as pl
from jax.experimental.pallas import tpu as pltpu
from jax.experimental.pallas import tpu_sc as plsc

_L = 16
_B = 1024
_NC = 2
_NS = 16
_SL = 8


def _digits(key):
    d1 = jnp.minimum(lax.shift_right_logical(key, 20), _B - 1)
    d2 = jnp.bitwise_and(lax.shift_right_logical(key, 10), _B - 1)
    d3 = jnp.bitwise_and(key, _B - 1)
    return d1, d2, d3


def _level_digit(w16, bsel, lvl):
    key = plsc.bitcast(w16, jnp.int32)
    d1, d2, d3 = _digits(key)
    if lvl == 0:
        return d1, w16
    if lvl == 1:
        act = d1 == bsel[0]
        return d2, jnp.where(act, w16, 0.0)
    act = (d1 == bsel[0]) & (d2 == bsel[1])
    return d3, jnp.where(act, w16, 0.0)


def _policy_sc(attn, thresh16):
    batch, n = attn.shape
    spw = n // _SL
    nchunk = spw // _L
    rows_per_core = batch // _NC
    mesh = plsc.VectorSubcoreMesh(
        core_axis_name="c", subcore_axis_name="s", num_cores=_NC,
        num_subcores=_NS)

    @functools.partial(
        pl.kernel,
        out_type=jax.ShapeDtypeStruct((batch, n), jnp.float32),
        mesh=mesh,
        compiler_params=pltpu.CompilerParams(needs_layout_passes=False),
        scratch_types=[
            pltpu.VMEM((spw,), jnp.float32),
            pltpu.VMEM((_L * _B,), jnp.float32),
            pltpu.VMEM((_B,), jnp.float32),
            pltpu.VMEM((spw,), jnp.float32),
            pltpu.VMEM((_L,), jnp.float32),
            pltpu.VMEM((_L,), jnp.float32),
            pltpu.VMEM((_SL, _L), jnp.float32),
            pltpu.VMEM((_SL, _B), jnp.float32),
            pltpu.VMEM_SHARED((rows_per_core, 3, _SL, _B), jnp.float32),
            pltpu.VMEM_SHARED((rows_per_core, _SL, _L), jnp.float32),
        ],
    )
    def k(attn_hbm, th_hbm, out_hbm, w_v, h_v, c_v, o_v, t_v, tmp_v, tb_v,
          r_v, sh_h, sh_t):
        cid = lax.axis_index("c")
        sid = lax.axis_index("s")
        rin = sid // _SL
        row = cid * rows_per_core + rin
        sl = sid % _SL
        off = sl * spw

        pltpu.sync_copy(attn_hbm.at[row, pl.ds(off, spw)], w_v)
        pltpu.sync_copy(th_hbm, t_v)
        thv = t_v[...]
        lane = lax.iota(jnp.int32, _L)
        idx15 = jnp.full((_L,), _L - 1, jnp.int32)
        zero16 = jnp.zeros((_L,), jnp.float32)
        slv = jnp.full((_L,), sl, jnp.int32)

        def zh(g, _):
            h_v[pl.ds(g * _L, _L)] = jnp.zeros((_L,), jnp.float32)
            return 0
        lax.fori_loop(0, _L * _B // _L, zh, 0)

        base = zero16
        bsel = []
        for lvl in range(3):
            def hbody(i, _, lvl=lvl):
                w16 = w_v[pl.ds(i * _L, _L)]
                d, val = _level_digit(w16, bsel, lvl)
                plsc.addupdate_scatter(h_v, [lane * _B + d], val)
                return 0
            lax.fori_loop(0, nchunk, hbody, 0)

            def rbody(g, _):
                acc = h_v[pl.ds(g * _L, _L)]
                for l in range(1, _L):
                    acc = acc + h_v[pl.ds(l * _B + g * _L, _L)]
                c_v[pl.ds(g * _L, _L)] = acc
                return 0
            lax.fori_loop(0, _B // _L, rbody, 0)
            pltpu.sync_copy(c_v, sh_h.at[rin, lvl, sl])

            def clbody(i, _, lvl=lvl):
                w16 = w_v[pl.ds(i * _L, _L)]
                d, _val = _level_digit(w16, bsel, lvl)
                plsc.store_scatter(h_v, [lane * _B + d],
                                   jnp.zeros((_L,), jnp.float32))
                return 0
            lax.fori_loop(0, nchunk, clbody, 0)
            plsc.subcore_barrier()

            pltpu.sync_copy(sh_h.at[rin, lvl], r_v)

            def cbody(g, carry):
                acc = r_v[0, pl.ds(g * _L, _L)]
                for kk in range(1, _SL):
                    acc = acc + r_v[kk, pl.ds(g * _L, _L)]
                cs = plsc.cumsum(acc) + carry
                c_v[pl.ds(g * _L, _L)] = cs
                return plsc.load_gather(
                    c_v, [jnp.full((_L,), g * _L + _L - 1, jnp.int32)])
            total = lax.fori_loop(0, _B // _L, cbody, zero16)

            remv = total - (thv - base)

            def kbody(g, cntl):
                c16 = c_v[pl.ds(g * _L, _L)]
                return cntl + (c16 <= remv).astype(jnp.float32)
            cntl = lax.fori_loop(0, _B // _L, kbody, zero16)
            tmp_v[...] = plsc.cumsum(cntl)
            cnt = plsc.load_gather(tmp_v, [idx15])
            bstar = jnp.minimum(cnt.astype(jnp.int32), _B - 1)

            cb = plsc.load_gather(c_v, [bstar])
            base = base + (total - cb)
            bsel.append(bstar)

        keystar = jnp.bitwise_or(
            jnp.bitwise_or(lax.shift_left(bsel[0], 20),
                           lax.shift_left(bsel[1], 10)), bsel[2])
        wstar16 = plsc.bitcast(keystar, jnp.float32)

        def tbody(i, acc):
            w16 = w_v[pl.ds(i * _L, _L)]
            key = plsc.bitcast(w16, jnp.int32)
            d1, d2, d3 = _digits(key)
            e3 = (d1 == bsel[0]) & (d2 == bsel[1]) & (d3 == bsel[2])
            return acc + e3.astype(jnp.float32)
        tcl = lax.fori_loop(0, nchunk, tbody, zero16)
        tmp_v[...] = plsc.cumsum(tcl)
        myties = plsc.load_gather(tmp_v, [idx15])
        tmp_v[...] = myties
        pltpu.sync_copy(tmp_v, sh_t.at[rin, sl])
        plsc.subcore_barrier()
        pltpu.sync_copy(sh_t.at[rin], tb_v)
        toff = zero16
        for kk in range(_SL):
            ck = tb_v[kk, :]
            toff = toff + jnp.where(jnp.full((_L,), kk, jnp.int32) < slv,
                                    ck, 0.0)

        def fbody(i, tcar):
            w16 = w_v[pl.ds(i * _L, _L)]
            key = plsc.bitcast(w16, jnp.int32)
            d1, d2, d3 = _digits(key)
            e1 = d1 == bsel[0]
            e2 = e1 & (d2 == bsel[1])
            e3 = e2 & (d3 == bsel[2])
            a3f = e3.astype(jnp.float32)
            t16 = plsc.cumsum(a3f) + tcar
            sel = ((d1 > bsel[0])
                   | (e1 & (d2 > bsel[1]))
                   | (e2 & (d3 > bsel[2]))
                   | (e3 & (base + t16 * wstar16 < thv)))
            o_v[pl.ds(i * _L, _L)] = 1.0 - sel.astype(jnp.float32)
            tmp_v[...] = t16
            return plsc.load_gather(tmp_v, [idx15])
        lax.fori_loop(0, nchunk, fbody, toff)
        pltpu.sync_copy(o_v, out_hbm.at[row, pl.ds(off, spw)])

    return k(attn, thresh16)


def _copy_tc(x2):
    m, d = x2.shape
    tb = 4096

    def body(x_ref, y_ref):
        y_ref[...] = x_ref[...]

    return pl.pallas_call(
        body,
        grid=(m // tb,),
        in_specs=[pl.BlockSpec((tb, d), lambda g: (g, 0))],
        out_specs=pl.BlockSpec((tb, d), lambda g: (g, 0)),
        out_shape=jax.ShapeDtypeStruct((m, d), x2.dtype),
    )(x2)


def _fixup_tc(y2, pol2):
    m, d = y2.shape
    tb = 1024
    nb = m // tb

    def body(y_in, p_ref, y_out, vbuf, sem):
        for i in range(nb):
            pol = p_ref[pl.ds(i * tb, tb), :]

            @pl.when(jnp.min(pol) < 0.5)
            def _(i=i, pol=pol):
                cp_in = pltpu.make_async_copy(
                    y_in.at[pl.ds(i * tb, tb), :], vbuf, sem)
                cp_in.start()
                cp_in.wait()
                vbuf[...] = vbuf[...] * pol
                cp_out = pltpu.make_async_copy(
                    vbuf, y_out.at[pl.ds(i * tb, tb), :], sem)
                cp_out.start()
                cp_out.wait()

    return pl.pallas_call(
        body,
        in_specs=[
            pl.BlockSpec(memory_space=pl.ANY),
            pl.BlockSpec(memory_space=pltpu.VMEM),
        ],
        out_specs=pl.BlockSpec(memory_space=pl.ANY),
        out_shape=jax.ShapeDtypeStruct((m, d), y2.dtype),
        scratch_shapes=[
            pltpu.VMEM((tb, d), jnp.float32),
            pltpu.SemaphoreType.DMA,
        ],
        input_output_aliases={0: 0},
    )(y2, pol2)


def kernel(x, attn_weight, thresh):
    batch, n, d = x.shape
    th16 = jnp.broadcast_to(thresh.astype(jnp.float32), (_L,))
    policy = _policy_sc(attn_weight, th16)
    y0 = _copy_tc(x.reshape(batch * n, d))
    y2 = _fixup_tc(y0, policy.reshape(batch * n, 1))
    return (y2.reshape(batch, n, d), policy)
